# Initial kernel scaffold; baseline (speedup 1.0000x reference)
import jax
import jax.numpy as jnp
from jax.experimental import pallas as pl


def kernel(x, qkv):
    raise NotImplementedError("write your pallas kernel here")


# speedup vs baseline: 4.8354x; 4.8354x over previous
import functools

import jax
import jax.numpy as jnp
from jax import lax
from jax.experimental import pallas as pl
from jax.experimental.pallas import tpu as pltpu
from jax.experimental.pallas import tpu_sc as plsc

_DIM = 384
_NH = 12
_HD = 32
_H = 32
_W = 32
_RS = 4
_RN = 4
_N = 1 + _H * _W
_WP = _W - _RS + 1
_NEG = jnp.float32(-jnp.inf)


def _tc_weights_body(q_ref, k_ref, out_ref):
    q = q_ref[0]
    k = k_ref[0]
    qc = q[0:1, :]
    kc = k[0:1, :]
    d_io = lax.broadcasted_iota(jnp.int32, (_NH, _DIM), 1)
    h_io = lax.broadcasted_iota(jnp.int32, (_NH, _DIM), 0)
    msk = d_io // _HD == h_io
    qsel = jnp.where(msk, jnp.broadcast_to(qc, (_NH, _DIM)), 0.0)
    ksel = jnp.where(msk, jnp.broadcast_to(kc, (_NH, _DIM)), 0.0)
    dn = (((1,), (1,)), ((), ()))
    w1 = lax.dot_general(k.astype(jnp.bfloat16), qsel.astype(jnp.bfloat16),
                         dn, preferred_element_type=jnp.float32)
    w2 = lax.dot_general(q.astype(jnp.bfloat16), ksel.astype(jnp.bfloat16),
                         dn, preferred_element_type=jnp.float32)

    def _softmax0(w):
        m = jnp.max(w, axis=0, keepdims=True)
        e = jnp.exp(w - m)
        return e / jnp.sum(e, axis=0, keepdims=True)

    aw = _softmax0(w1) * _softmax0(w2)
    hs = aw[:, 0:1]
    for h in range(1, _NH):
        hs = hs + aw[:, h:h + 1]
    wv = hs / float(_NH)
    out_ref[0] = wv[1:, :]


def _tc_weights(qkv):
    b = qkv.shape[0]
    return pl.pallas_call(
        _tc_weights_body,
        grid=(b,),
        in_specs=[
            pl.BlockSpec((1, _N, _DIM), lambda i: (i, 0, 0)),
            pl.BlockSpec((1, _N, _DIM), lambda i: (i, 0, 1)),
        ],
        out_specs=pl.BlockSpec((1, _H * _W, 1), lambda i: (i, 0, 0)),
        out_shape=jax.ShapeDtypeStruct((b, _H * _W, 1), jnp.float32),
        compiler_params=pltpu.CompilerParams(
            dimension_semantics=("arbitrary",),
        ),
    )(qkv, qkv)


_L = 16


def _sc_worker_id():
    return lax.axis_index("s") * 2 + lax.axis_index("c")


def _sc_body(w_hbm, x_hbm, out_hbm, roi_hbm,
             w_v, h_v, pooled_v, idx_v, rows_v, out_v, roi_v, sem):
    bidx = _sc_worker_id()
    iota = lax.broadcasted_iota(jnp.int32, (_L,), 0)
    zero16 = jnp.zeros((_L,), jnp.float32)

    pltpu.sync_copy(w_hbm.at[bidx], w_v.at[pl.ds(0, _H * _W)])
    w_v[pl.ds(_H * _W, _L)] = zero16
    w_v[pl.ds(_H * _W + _L, _L)] = zero16

    def _hpool_row(y, _):
        base = y * _W
        for c in range(2):
            off = base + c * _L
            s = ((w_v[pl.ds(off, _L)] + w_v[pl.ds(off + 1, _L)])
                 + w_v[pl.ds(off + 2, _L)]) + w_v[pl.ds(off + 3, _L)]
            h_v[pl.ds(off, _L)] = s
        return 0

    lax.fori_loop(0, _H, _hpool_row, 0)

    def _vpool_row(i, _):
        for c in range(2):
            off = i * _W + c * _L
            s = ((h_v[pl.ds(off, _L)] + h_v[pl.ds(off + _W, _L)])
                 + h_v[pl.ds(off + 2 * _W, _L)]) + h_v[pl.ds(off + 3 * _W, _L)]
            col = c * _L + iota
            s = jnp.where(col <= _WP - 1, s, _NEG)
            pooled_v[pl.ds(off, _L)] = s
        return 0

    lax.fori_loop(0, _WP, _vpool_row, 0)

    nchunk = (_WP * _W) // _L
    picks = []
    for _r in range(_RN):
        def _scan(t, carry):
            bv, bi = carry
            ch = pooled_v[pl.ds(t * _L, _L)]
            ix = t * _L + iota
            better = ch > bv
            return (jnp.where(better, ch, bv), jnp.where(better, ix, bi))

        bv, bi = lax.fori_loop(
            0, nchunk, _scan,
            (jnp.full((_L,), _NEG, jnp.float32), jnp.zeros((_L,), jnp.int32)))
        mx = jnp.max(bv)
        am = jnp.min(jnp.where(bv == mx, bi, jnp.int32(2 ** 30)))
        i_r = am // _W
        j_r = am % _W
        picks.append((i_r, j_r))
        for di in range(-2, 3):
            row = i_r + di
            ok = jnp.logical_and(row >= 0, row <= _WP - 1)
            row_c = jnp.clip(row, 0, _WP - 1)
            si = _RS - abs(di)
            for c in range(2):
                off = row_c * _W + c * _L
                v = pooled_v[pl.ds(off, _L)]
                dj = jnp.abs(c * _L + iota - j_r)
                inter = si * jnp.maximum(_RS - dj, 0)
                supp = jnp.logical_and(inter >= 8, ok)
                pooled_v[pl.ds(off, _L)] = jnp.where(supp, _NEG, v)

    for c in range(2):
        s = c * _L + iota
        rr = s // 8
        t = s % 8
        h_sel = jnp.where(rr == 0, picks[0][0],
                 jnp.where(rr == 1, picks[1][0],
                  jnp.where(rr == 2, picks[2][0], picks[3][0])))
        w_sel = jnp.where(rr == 0, picks[0][1],
                 jnp.where(rr == 1, picks[1][1],
                  jnp.where(rr == 2, picks[2][1], picks[3][1])))
        val = jnp.where(t == 0, bidx,
               jnp.where(t == 1, h_sel,
                jnp.where(t == 2, w_sel,
                 jnp.where(t == 3, h_sel + _RS,
                  jnp.where(t == 4, w_sel + _RS, 0)))))
        roi_v[pl.ds(c * _L, _L)] = val.astype(jnp.float32)
    pltpu.sync_copy(roi_v, roi_hbm.at[bidx])

    tok0 = bidx * _N
    for r in range(_RN):
        i_r, j_r = picks[r]
        for c in range(2):
            s = c * _L + iota
            i5 = s // 5
            j5 = s % 5
            yy = j_r + i5
            xx = i_r + j5
            valid = jnp.logical_and(
                s < 25, jnp.logical_and(yy <= _H - 1, xx <= _W - 1))
            tok = jnp.where(valid, tok0 + 1 + yy * _W + xx, tok0)
            idx_v[pl.ds(r * 2 * _L + c * _L, _L)] = tok
    pltpu.async_copy(x_hbm.at[idx_v], rows_v, sem).wait()

    pens = []
    for r in range(_RN):
        i_r, j_r = picks[r]
        pens.append((jnp.where(j_r == _WP - 1, _NEG, jnp.float32(0.0)),
                     jnp.where(i_r == _WP - 1, _NEG, jnp.float32(0.0))))

    def _bins(t, _):
        off = t * _L
        out_v[0, pl.ds(off, _L)] = rows_v[25, pl.ds(off, _L)]
        for r in range(_RN):
            pen_i4, pen_j4 = pens[r]
            ch = []
            for s in range(25):
                v = rows_v[r * 2 * _L + s, pl.ds(off, _L)]
                if s // 5 == 4:
                    v = v + pen_i4
                if s % 5 == 4:
                    v = v + pen_j4
                ch.append(v)
            for p in range(_RS):
                hm = [jnp.maximum(ch[(p) * 5 + q], ch[(p + 1) * 5 + q])
                      for q in range(5)]
                for q in range(_RS):
                    val = jnp.maximum(hm[q], hm[q + 1])
                    out_v[1 + r * 16 + p * 4 + q, pl.ds(off, _L)] = val
        return 0

    lax.fori_loop(0, _DIM // _L, _bins, 0)
    pltpu.sync_copy(out_v, out_hbm.at[bidx])


def _sc_stage(w2d, x_flat):
    b = w2d.shape[0]
    mesh = plsc.VectorSubcoreMesh(core_axis_name="c", subcore_axis_name="s")
    fn = functools.partial(
        pl.kernel,
        mesh=mesh,
        out_type=[
            jax.ShapeDtypeStruct((b, 1 + _RN * 16, _DIM), jnp.float32),
            jax.ShapeDtypeStruct((b, _RN * 8), jnp.float32),
        ],
        scratch_types=[
            pltpu.VMEM((_H * _W + 2 * _L,), jnp.float32),
            pltpu.VMEM((_H * _W,), jnp.float32),
            pltpu.VMEM((_WP * _W,), jnp.float32),
            pltpu.VMEM((_RN * 2 * _L,), jnp.int32),
            pltpu.VMEM((_RN * 2 * _L, _DIM), jnp.float32),
            pltpu.VMEM((1 + _RN * 16, _DIM), jnp.float32),
            pltpu.VMEM((2 * _L,), jnp.float32),
            pltpu.SemaphoreType.DMA,
        ],
        compiler_params=pltpu.CompilerParams(needs_layout_passes=False),
    )(_sc_body)
    return fn(w2d, x_flat)


def kernel(x, qkv):
    b = x.shape[0]
    w2d = _tc_weights(qkv)[:, :, 0]
    x_flat = x.reshape(b * _N, _DIM)
    out_flat, roi_pad = _sc_stage(w2d, x_flat)
    rois = roi_pad.reshape(b, _RN, 8)[:, :, :5].reshape(b * _RN, 5)
    return out_flat, rois


# speedup vs baseline: 5.6918x; 1.1771x over previous
import functools

import jax
import jax.numpy as jnp
from jax import lax
from jax.experimental import pallas as pl
from jax.experimental.pallas import tpu as pltpu
from jax.experimental.pallas import tpu_sc as plsc

_DIM = 384
_NH = 12
_HD = 32
_H = 32
_W = 32
_RS = 4
_RN = 4
_N = 1 + _H * _W
_WP = _W - _RS + 1
_NEG = jnp.float32(-jnp.inf)


def _tc_weights_body(q_ref, k_ref, out_ref):
    q = q_ref[0]
    k = k_ref[0]
    qc = q[0:1, :]
    kc = k[0:1, :]
    d_io = lax.broadcasted_iota(jnp.int32, (_NH, _DIM), 1)
    h_io = lax.broadcasted_iota(jnp.int32, (_NH, _DIM), 0)
    msk = d_io // _HD == h_io
    qsel = jnp.where(msk, jnp.broadcast_to(qc, (_NH, _DIM)), 0.0)
    ksel = jnp.where(msk, jnp.broadcast_to(kc, (_NH, _DIM)), 0.0)
    dn = (((1,), (1,)), ((), ()))
    w1 = lax.dot_general(k.astype(jnp.bfloat16), qsel.astype(jnp.bfloat16),
                         dn, preferred_element_type=jnp.float32)
    w2 = lax.dot_general(q.astype(jnp.bfloat16), ksel.astype(jnp.bfloat16),
                         dn, preferred_element_type=jnp.float32)

    def _softmax0(w):
        m = jnp.max(w, axis=0, keepdims=True)
        e = jnp.exp(w - m)
        return e / jnp.sum(e, axis=0, keepdims=True)

    aw = _softmax0(w1) * _softmax0(w2)
    hs = aw[:, 0:1]
    for h in range(1, _NH):
        hs = hs + aw[:, h:h + 1]
    wv = hs / float(_NH)
    out_ref[0] = wv[1:, :]


def _tc_weights(qkv):
    b = qkv.shape[0]
    return pl.pallas_call(
        _tc_weights_body,
        grid=(b,),
        in_specs=[
            pl.BlockSpec((1, _N, _DIM), lambda i: (i, 0, 0)),
            pl.BlockSpec((1, _N, _DIM), lambda i: (i, 0, 1)),
        ],
        out_specs=pl.BlockSpec((1, _H * _W, 1), lambda i: (i, 0, 0)),
        out_shape=jax.ShapeDtypeStruct((b, _H * _W, 1), jnp.float32),
        compiler_params=pltpu.CompilerParams(
            dimension_semantics=("arbitrary",),
        ),
    )(qkv, qkv)


_L = 16


def _sc_worker_id():
    return lax.axis_index("s") * 2 + lax.axis_index("c")


def _sc_body(w_hbm, x_hbm, out_hbm, roi_hbm,
             w_v, h_v, pooled_v, rows_v, out_v, roi_v, sem):
    bidx = _sc_worker_id()
    iota = lax.broadcasted_iota(jnp.int32, (_L,), 0)
    zero16 = jnp.zeros((_L,), jnp.float32)

    pltpu.sync_copy(w_hbm.at[bidx], w_v.at[pl.ds(0, _H * _W)])
    w_v[pl.ds(_H * _W, _L)] = zero16
    w_v[pl.ds(_H * _W + _L, _L)] = zero16

    def _hpool_row(y, _):
        base = y * _W
        for c in range(2):
            off = base + c * _L
            s = ((w_v[pl.ds(off, _L)] + w_v[pl.ds(off + 1, _L)])
                 + w_v[pl.ds(off + 2, _L)]) + w_v[pl.ds(off + 3, _L)]
            h_v[pl.ds(off, _L)] = s
        return 0

    lax.fori_loop(0, _H, _hpool_row, 0)

    def _vpool_row(i, _):
        for c in range(2):
            off = i * _W + c * _L
            s = ((h_v[pl.ds(off, _L)] + h_v[pl.ds(off + _W, _L)])
                 + h_v[pl.ds(off + 2 * _W, _L)]) + h_v[pl.ds(off + 3 * _W, _L)]
            col = c * _L + iota
            s = jnp.where(col <= _WP - 1, s, _NEG)
            pooled_v[pl.ds(off, _L)] = s
        return 0

    lax.fori_loop(0, _WP, _vpool_row, 0)

    nchunk = (_WP * _W) // _L
    picks = []
    for _r in range(_RN):
        def _scan(t, carry):
            bv, bi = carry
            ch = pooled_v[pl.ds(t * _L, _L)]
            ix = t * _L + iota
            better = ch > bv
            return (jnp.where(better, ch, bv), jnp.where(better, ix, bi))

        bv, bi = lax.fori_loop(
            0, nchunk, _scan,
            (jnp.full((_L,), _NEG, jnp.float32), jnp.zeros((_L,), jnp.int32)))
        mx = jnp.max(bv)
        am = jnp.min(jnp.where(bv == mx, bi, jnp.int32(2 ** 30)))
        i_r = am // _W
        j_r = am % _W
        picks.append((i_r, j_r))
        for di in range(-2, 3):
            row = i_r + di
            ok = jnp.logical_and(row >= 0, row <= _WP - 1)
            row_c = jnp.clip(row, 0, _WP - 1)
            si = _RS - abs(di)
            for c in range(2):
                off = row_c * _W + c * _L
                v = pooled_v[pl.ds(off, _L)]
                dj = jnp.abs(c * _L + iota - j_r)
                inter = si * jnp.maximum(_RS - dj, 0)
                supp = jnp.logical_and(inter >= 8, ok)
                pooled_v[pl.ds(off, _L)] = jnp.where(supp, _NEG, v)

    for c in range(2):
        s = c * _L + iota
        rr = s // 8
        t = s % 8
        h_sel = jnp.where(rr == 0, picks[0][0],
                 jnp.where(rr == 1, picks[1][0],
                  jnp.where(rr == 2, picks[2][0], picks[3][0])))
        w_sel = jnp.where(rr == 0, picks[0][1],
                 jnp.where(rr == 1, picks[1][1],
                  jnp.where(rr == 2, picks[2][1], picks[3][1])))
        val = jnp.where(t == 0, bidx,
               jnp.where(t == 1, h_sel,
                jnp.where(t == 2, w_sel,
                 jnp.where(t == 3, h_sel + _RS,
                  jnp.where(t == 4, w_sel + _RS, 0)))))
        roi_v[pl.ds(c * _L, _L)] = val.astype(jnp.float32)
    pltpu.sync_copy(roi_v, roi_hbm.at[bidx])

    pltpu.sync_copy(x_hbm.at[bidx, pl.ds(0, 1)], out_v.at[pl.ds(0, 1)])
    for r in range(_RN):
        i_r, j_r = picks[r]
        descs = []
        for i in range(5):
            for j in range(5):
                tok = jnp.minimum(1 + (j_r + i) * _W + (i_r + j), _N - 1)
                descs.append(pltpu.async_copy(
                    x_hbm.at[bidx, pl.ds(tok, 1)],
                    rows_v.at[pl.ds(r * 2 * _L + i * 5 + j, 1)], sem))
        for d in descs:
            d.wait()

    pens = []
    for r in range(_RN):
        i_r, j_r = picks[r]
        pens.append((jnp.where(j_r == _WP - 1, _NEG, jnp.float32(0.0)),
                     jnp.where(i_r == _WP - 1, _NEG, jnp.float32(0.0))))

    def _bins(t, _):
        off = t * _L
        for r in range(_RN):
            pen_i4, pen_j4 = pens[r]
            ch = []
            for s in range(25):
                v = rows_v[r * 2 * _L + s, pl.ds(off, _L)]
                if s // 5 == 4:
                    v = v + pen_i4
                if s % 5 == 4:
                    v = v + pen_j4
                ch.append(v)
            for p in range(_RS):
                hm = [jnp.maximum(ch[(p) * 5 + q], ch[(p + 1) * 5 + q])
                      for q in range(5)]
                for q in range(_RS):
                    val = jnp.maximum(hm[q], hm[q + 1])
                    out_v[1 + r * 16 + p * 4 + q, pl.ds(off, _L)] = val
        return 0

    lax.fori_loop(0, _DIM // _L, _bins, 0)
    pltpu.sync_copy(out_v, out_hbm.at[bidx])


def _sc_stage(w2d, x_flat):
    b = w2d.shape[0]
    mesh = plsc.VectorSubcoreMesh(core_axis_name="c", subcore_axis_name="s")
    fn = functools.partial(
        pl.kernel,
        mesh=mesh,
        out_type=[
            jax.ShapeDtypeStruct((b, 1 + _RN * 16, _DIM), jnp.float32),
            jax.ShapeDtypeStruct((b, _RN * 8), jnp.float32),
        ],
        scratch_types=[
            pltpu.VMEM((_H * _W + 2 * _L,), jnp.float32),
            pltpu.VMEM((_H * _W,), jnp.float32),
            pltpu.VMEM((_WP * _W,), jnp.float32),
            pltpu.VMEM((_RN * 2 * _L, _DIM), jnp.float32),
            pltpu.VMEM((1 + _RN * 16, _DIM), jnp.float32),
            pltpu.VMEM((2 * _L,), jnp.float32),
            pltpu.SemaphoreType.DMA,
        ],
        compiler_params=pltpu.CompilerParams(needs_layout_passes=False),
    )(_sc_body)
    return fn(w2d, x_flat)


def kernel(x, qkv):
    b = x.shape[0]
    w2d = _tc_weights(qkv)[:, :, 0]
    out_flat, roi_pad = _sc_stage(w2d, x)
    rois = roi_pad.reshape(b, _RN, 8)[:, :, :5].reshape(b * _RN, 5)
    return out_flat, rois


# speedup vs baseline: 5.7439x; 1.0091x over previous
import functools

import jax
import jax.numpy as jnp
from jax import lax
from jax.experimental import pallas as pl
from jax.experimental.pallas import tpu as pltpu
from jax.experimental.pallas import tpu_sc as plsc

_DIM = 384
_NH = 12
_HD = 32
_H = 32
_W = 32
_RS = 4
_RN = 4
_N = 1 + _H * _W
_WP = _W - _RS + 1
_NEG = jnp.float32(-jnp.inf)


def _tc_weights_body(q_ref, k_ref, out_ref):
    q = q_ref[0]
    k = k_ref[0]
    qc = q[0:1, :]
    kc = k[0:1, :]
    d_io = lax.broadcasted_iota(jnp.int32, (_NH, _DIM), 1)
    h_io = lax.broadcasted_iota(jnp.int32, (_NH, _DIM), 0)
    msk = d_io // _HD == h_io
    qsel = jnp.where(msk, jnp.broadcast_to(qc, (_NH, _DIM)), 0.0)
    ksel = jnp.where(msk, jnp.broadcast_to(kc, (_NH, _DIM)), 0.0)
    dn = (((1,), (1,)), ((), ()))
    w1 = lax.dot_general(k.astype(jnp.bfloat16), qsel.astype(jnp.bfloat16),
                         dn, preferred_element_type=jnp.float32)
    w2 = lax.dot_general(q.astype(jnp.bfloat16), ksel.astype(jnp.bfloat16),
                         dn, preferred_element_type=jnp.float32)

    def _softmax0(w):
        m = jnp.max(w, axis=0, keepdims=True)
        e = jnp.exp(w - m)
        return e / jnp.sum(e, axis=0, keepdims=True)

    aw = _softmax0(w1) * _softmax0(w2)
    hs = aw[:, 0:1]
    for h in range(1, _NH):
        hs = hs + aw[:, h:h + 1]
    wv = hs / float(_NH)
    out_ref[0] = jnp.transpose(wv[1:, :])


def _tc_weights(qkv):
    b = qkv.shape[0]
    return pl.pallas_call(
        _tc_weights_body,
        grid=(b,),
        in_specs=[
            pl.BlockSpec((1, _N, _DIM), lambda i: (i, 0, 0)),
            pl.BlockSpec((1, _N, _DIM), lambda i: (i, 0, 1)),
        ],
        out_specs=pl.BlockSpec((1, 1, _H * _W), lambda i: (i, 0, 0)),
        out_shape=jax.ShapeDtypeStruct((b, 1, _H * _W), jnp.float32),
        compiler_params=pltpu.CompilerParams(
            dimension_semantics=("arbitrary",),
        ),
    )(qkv, qkv)


_L = 16


def _sc_worker_id():
    return lax.axis_index("s") * 2 + lax.axis_index("c")


def _sc_body(w_hbm, x_hbm, out_hbm, roi_hbm,
             w_v, h_v, pooled_v, rows_v, out_v, roi_v, sem):
    bidx = _sc_worker_id()
    iota = lax.broadcasted_iota(jnp.int32, (_L,), 0)
    zero16 = jnp.zeros((_L,), jnp.float32)

    pltpu.sync_copy(w_hbm.at[bidx], w_v.at[pl.ds(0, _H * _W)])
    w_v[pl.ds(_H * _W, _L)] = zero16
    w_v[pl.ds(_H * _W + _L, _L)] = zero16

    def _hpool_row(y, _):
        base = y * _W
        for c in range(2):
            off = base + c * _L
            s = ((w_v[pl.ds(off, _L)] + w_v[pl.ds(off + 1, _L)])
                 + w_v[pl.ds(off + 2, _L)]) + w_v[pl.ds(off + 3, _L)]
            h_v[pl.ds(off, _L)] = s
        return 0

    lax.fori_loop(0, _H, _hpool_row, 0)

    def _vpool_row(i, _):
        for c in range(2):
            off = i * _W + c * _L
            s = ((h_v[pl.ds(off, _L)] + h_v[pl.ds(off + _W, _L)])
                 + h_v[pl.ds(off + 2 * _W, _L)]) + h_v[pl.ds(off + 3 * _W, _L)]
            col = c * _L + iota
            s = jnp.where(col <= _WP - 1, s, _NEG)
            pooled_v[pl.ds(off, _L)] = s
        return 0

    lax.fori_loop(0, _WP, _vpool_row, 0)

    nchunk = (_WP * _W) // _L
    picks = []
    for _r in range(_RN):
        def _scan(t, carry):
            bv, bi = carry
            ch = pooled_v[pl.ds(t * _L, _L)]
            ix = t * _L + iota
            better = ch > bv
            return (jnp.where(better, ch, bv), jnp.where(better, ix, bi))

        bv, bi = lax.fori_loop(
            0, nchunk, _scan,
            (jnp.full((_L,), _NEG, jnp.float32), jnp.zeros((_L,), jnp.int32)))
        mx = jnp.max(bv)
        am = jnp.min(jnp.where(bv == mx, bi, jnp.int32(2 ** 30)))
        i_r = am // _W
        j_r = am % _W
        picks.append((i_r, j_r))
        for di in range(-2, 3):
            row = i_r + di
            ok = jnp.logical_and(row >= 0, row <= _WP - 1)
            row_c = jnp.clip(row, 0, _WP - 1)
            si = _RS - abs(di)
            for c in range(2):
                off = row_c * _W + c * _L
                v = pooled_v[pl.ds(off, _L)]
                dj = jnp.abs(c * _L + iota - j_r)
                inter = si * jnp.maximum(_RS - dj, 0)
                supp = jnp.logical_and(inter >= 8, ok)
                pooled_v[pl.ds(off, _L)] = jnp.where(supp, _NEG, v)

    for c in range(2):
        s = c * _L + iota
        rr = s // 8
        t = s % 8
        h_sel = jnp.where(rr == 0, picks[0][0],
                 jnp.where(rr == 1, picks[1][0],
                  jnp.where(rr == 2, picks[2][0], picks[3][0])))
        w_sel = jnp.where(rr == 0, picks[0][1],
                 jnp.where(rr == 1, picks[1][1],
                  jnp.where(rr == 2, picks[2][1], picks[3][1])))
        val = jnp.where(t == 0, bidx,
               jnp.where(t == 1, h_sel,
                jnp.where(t == 2, w_sel,
                 jnp.where(t == 3, h_sel + _RS,
                  jnp.where(t == 4, w_sel + _RS, 0)))))
        roi_v[pl.ds(c * _L, _L)] = val.astype(jnp.float32)
    pltpu.sync_copy(roi_v, roi_hbm.at[bidx])

    pltpu.sync_copy(x_hbm.at[bidx, pl.ds(0, 1)], out_v.at[pl.ds(0, 1)])
    for r in range(_RN):
        i_r, j_r = picks[r]
        descs = []
        for i in range(5):
            for j in range(5):
                tok = jnp.minimum(1 + (j_r + i) * _W + (i_r + j), _N - 1)
                descs.append(pltpu.async_copy(
                    x_hbm.at[bidx, pl.ds(tok, 1)],
                    rows_v.at[pl.ds(r * 2 * _L + i * 5 + j, 1)], sem))
        for d in descs:
            d.wait()

    pens = []
    for r in range(_RN):
        i_r, j_r = picks[r]
        pens.append((jnp.where(j_r == _WP - 1, _NEG, jnp.float32(0.0)),
                     jnp.where(i_r == _WP - 1, _NEG, jnp.float32(0.0))))

    def _bins(t, _):
        off = t * _L
        for r in range(_RN):
            pen_i4, pen_j4 = pens[r]
            ch = []
            for s in range(25):
                v = rows_v[r * 2 * _L + s, pl.ds(off, _L)]
                if s // 5 == 4:
                    v = v + pen_i4
                if s % 5 == 4:
                    v = v + pen_j4
                ch.append(v)
            for p in range(_RS):
                hm = [jnp.maximum(ch[(p) * 5 + q], ch[(p + 1) * 5 + q])
                      for q in range(5)]
                for q in range(_RS):
                    val = jnp.maximum(hm[q], hm[q + 1])
                    out_v[1 + r * 16 + p * 4 + q, pl.ds(off, _L)] = val
        return 0

    lax.fori_loop(0, _DIM // _L, _bins, 0)
    pltpu.sync_copy(out_v, out_hbm.at[bidx])


def _sc_stage(w2d, x_flat):
    b = w2d.shape[0]
    mesh = plsc.VectorSubcoreMesh(core_axis_name="c", subcore_axis_name="s")
    fn = functools.partial(
        pl.kernel,
        mesh=mesh,
        out_type=[
            jax.ShapeDtypeStruct((b, 1 + _RN * 16, _DIM), jnp.float32),
            jax.ShapeDtypeStruct((b, _RN * 8), jnp.float32),
        ],
        scratch_types=[
            pltpu.VMEM((_H * _W + 2 * _L,), jnp.float32),
            pltpu.VMEM((_H * _W,), jnp.float32),
            pltpu.VMEM((_WP * _W,), jnp.float32),
            pltpu.VMEM((_RN * 2 * _L, _DIM), jnp.float32),
            pltpu.VMEM((1 + _RN * 16, _DIM), jnp.float32),
            pltpu.VMEM((2 * _L,), jnp.float32),
            pltpu.SemaphoreType.DMA,
        ],
        compiler_params=pltpu.CompilerParams(needs_layout_passes=False),
    )(_sc_body)
    return fn(w2d, x_flat)


def kernel(x, qkv):
    b = x.shape[0]
    w2d = _tc_weights(qkv)[:, 0, :]
    out_flat, roi_pad = _sc_stage(w2d, x)
    rois = roi_pad.reshape(b, _RN, 8)[:, :, :5].reshape(b * _RN, 5)
    return out_flat, rois
